# Initial kernel scaffold; baseline (speedup 1.0000x reference)
import jax
import jax.numpy as jnp
from jax.experimental import pallas as pl


def kernel(x, ln_g0, ln_b0, Wp0, bp0, ln_g1, ln_b1, Wp1, bp1, ln_g2, ln_b2, Wp2, bp2, Wg1, bg1, Wg2, bg2, Wu1, bu1, Wu2, bu2, Wc1, bc1, bn_g, bn_b, bn_m, bn_v, Wc2, bc2):
    raise NotImplementedError("write your pallas kernel here")


# speedup vs baseline: 7.6313x; 7.6313x over previous
import functools

import jax
import jax.numpy as jnp
from jax import lax
from jax.experimental import pallas as pl
from jax.experimental.pallas import tpu as pltpu, tpu_sc as plsc

B = 4096
H = 256
FUSED = 768
K = 8
BLK = 256
NBLK = B // BLK
BIG = 1e30


def _ln(v, g, b):
    mu = jnp.mean(v, axis=-1, keepdims=True)
    var = jnp.mean((v - mu) ** 2, axis=-1, keepdims=True)
    return (v - mu) / jnp.sqrt(var + 1e-5) * g + b


def _fuse_knn_body(x0, x1, x2, cc, ct, imr, imc,
                   lg0, lb0, wp0, bp0, lg1, lb1, wp1, bp1, lg2, lb2, wp2, bp2,
                   wg1, bg1, wg2, bg2,
                   fused_o, nidx_o, w_o, ent_o):
    i = pl.program_id(0)
    v0 = jnp.dot(_ln(x0[...], lg0[...], lb0[...]), wp0[...],
                 preferred_element_type=jnp.float32) + bp0[...]
    v1 = jnp.dot(_ln(x1[...], lg1[...], lb1[...]), wp1[...],
                 preferred_element_type=jnp.float32) + bp1[...]
    v2 = jnp.dot(_ln(x2[...], lg2[...], lb2[...]), wp2[...],
                 preferred_element_type=jnp.float32) + bp2[...]
    concat = jnp.concatenate([v0, v1, v2], axis=1)
    gh = jnp.maximum(jnp.dot(concat, wg1[...], preferred_element_type=jnp.float32)
                     + bg1[...], 0.0)
    gl = jnp.dot(gh, wg2[...], preferred_element_type=jnp.float32) + bg2[...]
    gm = jnp.max(gl, axis=1, keepdims=True)
    ge = jnp.exp(gl - gm)
    gp = ge / jnp.sum(ge, axis=1, keepdims=True)
    ent_blk = -jnp.sum(gp * jnp.log(gp + 1e-8))

    @pl.when(i == 0)
    def _():
        ent_o[0, 0] = 0.0
    ent_o[0, 0] += ent_blk * (0.01 / B)

    fused_o[...] = jnp.concatenate(
        [v0 * gp[:, 0:1], v1 * gp[:, 1:2], v2 * gp[:, 2:3]], axis=1)

    cb = cc[...]
    cxb = cb[:, 0:1]
    cyb = cb[:, 1:2]
    cx = ct[0:1, :]
    cy = ct[1:2, :]
    sqr = cxb * cxb + cyb * cyb
    sqc = cx * cx + cy * cy
    bf = lambda a: a.astype(jnp.bfloat16).astype(jnp.float32)
    prod = bf(cxb) * bf(cx) + bf(cyb) * bf(cy)
    d2 = sqr + sqc - 2.0 * prod
    rowid = i * BLK + lax.broadcasted_iota(jnp.int32, (BLK, 1), 0)
    colid = lax.broadcasted_iota(jnp.int32, (BLK, B), 1)
    diag = colid == rowid
    same = (imr[...] == imc[...]) & jnp.logical_not(diag)
    has_n = jnp.any(same, axis=1, keepdims=True)
    excl = diag | (has_n & jnp.logical_not(same))
    sel = jnp.where(excl, BIG, d2)
    vals = []
    idxs = []
    for _ in range(K):
        mn = jnp.min(sel, axis=1, keepdims=True)
        amin = jnp.min(jnp.where(sel == mn, colid, 2 ** 30),
                       axis=1, keepdims=True)
        vals.append(jnp.where(mn >= 1e29, -1e9,
                              -jnp.sqrt(jnp.maximum(mn, 1e-12))))
        idxs.append(amin)
        sel = jnp.where(colid == amin, BIG, sel)
    valsm = jnp.concatenate(vals, axis=1)
    nidx_o[...] = jnp.concatenate(idxs, axis=1)
    we = jnp.exp(valsm - valsm[:, 0:1])
    w = we / jnp.sum(we, axis=1, keepdims=True)
    w_o[...] = jnp.concatenate([w, jnp.zeros((BLK, 8), jnp.float32)], axis=1)


def _classifier_body(fused, agg, wu1, bu1, wu2, bu2, wc1, bc1,
                     bn_g, bn_b, bn_m, bn_v, wc2, bc2, out):
    h1 = jnp.maximum(jnp.dot(agg[...], wu1[...],
                             preferred_element_type=jnp.float32) + bu1[...], 0.0)
    upd = jnp.dot(h1, wu2[...], preferred_element_type=jnp.float32) + bu2[...]
    ctx = fused[...] + 0.5 * upd
    h = jnp.dot(ctx, wc1[...], preferred_element_type=jnp.float32) + bc1[...]
    h = (h - bn_m[...]) / jnp.sqrt(bn_v[...] + 1e-5) * bn_g[...] + bn_b[...]
    h = jnp.maximum(h, 0.0)
    out[...] = jnp.dot(h, wc2[...], preferred_element_type=jnp.float32) + bc2[...]


def _sc_agg_body(fused_hbm, idx_hbm, w_hbm, out_hbm,
                 idx_v, rows_v, w_v, agg_v, sem):
    wid = lax.axis_index("s") * 2 + lax.axis_index("c")
    base = wid * (B // 32)
    nchunk = (B // 32) // 16

    def chunk_body(c, carry):
        row0 = base + c * 16
        pltpu.sync_copy(idx_hbm.at[pl.ds(row0 * K, 16 * K)], idx_v)
        pltpu.async_copy(fused_hbm.at[idx_v], rows_v, sem).wait()
        pltpu.sync_copy(w_hbm.at[pl.ds(row0, 16)], w_v)

        def row_body(r, carry2):
            wrow = w_v[r, :]
            splats = [wrow.at[jnp.full((16,), k, jnp.int32)]
                      .get(mode="promise_in_bounds") for k in range(K)]

            def s_body(s, carry3):
                col = s * 16
                a = splats[0] * rows_v[r * K, pl.ds(col, 16)]
                for k in range(1, K):
                    a = a + splats[k] * rows_v[r * K + k, pl.ds(col, 16)]
                agg_v[r, pl.ds(col, 16)] = a
                return carry3

            return lax.fori_loop(0, FUSED // 16, s_body, carry2)

        lax.fori_loop(0, 16, row_body, 0)
        pltpu.sync_copy(agg_v, out_hbm.at[pl.ds(row0, 16)])
        return carry

    lax.fori_loop(0, nchunk, chunk_body, 0)


def _make_sc_agg():
    mesh = plsc.VectorSubcoreMesh(core_axis_name="c", subcore_axis_name="s")
    return pl.kernel(
        _sc_agg_body,
        out_type=jax.ShapeDtypeStruct((B, FUSED), jnp.float32),
        mesh=mesh,
        scratch_types=[
            pltpu.VMEM((16 * K,), jnp.int32),
            pltpu.VMEM((16 * K, FUSED), jnp.float32),
            pltpu.VMEM((16, 16), jnp.float32),
            pltpu.VMEM((16, FUSED), jnp.float32),
            pltpu.SemaphoreType.DMA,
        ],
    )


def _call_fuse_knn(x, ln_g0, ln_b0, Wp0, bp0, ln_g1, ln_b1, Wp1, bp1,
                   ln_g2, ln_b2, Wp2, bp2, Wg1, bg1, Wg2, bg2):
    x0 = x[:, 0:512]
    x1 = x[:, 512:1280]
    x2 = x[:, 1280:1536]
    cc = x[:, 1536:1538]
    ct = cc.T
    imc = x[:, 1540:1541]
    imr = imc.T
    r = lambda a: a.reshape(1, -1)
    wg2p = jnp.concatenate([Wg2, jnp.zeros((128, 5), jnp.float32)], axis=1)
    bg2p = jnp.concatenate([bg2, jnp.full((5,), -1e30, jnp.float32)]).reshape(1, 8)

    row_spec = lambda c: pl.BlockSpec((BLK, c), lambda i: (i, 0))
    full_spec = lambda rr, c: pl.BlockSpec((rr, c), lambda i: (0, 0))

    fused, nidx, w16, ent = pl.pallas_call(
        _fuse_knn_body,
        grid=(NBLK,),
        in_specs=[
            row_spec(512), row_spec(768), row_spec(256), row_spec(2),
            full_spec(2, B), full_spec(1, B), row_spec(1),
            full_spec(1, 512), full_spec(1, 512), full_spec(512, H), full_spec(1, H),
            full_spec(1, 768), full_spec(1, 768), full_spec(768, H), full_spec(1, H),
            full_spec(1, 256), full_spec(1, 256), full_spec(256, H), full_spec(1, H),
            full_spec(768, 128), full_spec(1, 128), full_spec(128, 8), full_spec(1, 8),
        ],
        out_specs=[
            row_spec(FUSED), row_spec(K), row_spec(16),
            pl.BlockSpec((1, 1), lambda i: (0, 0), memory_space=pltpu.SMEM),
        ],
        out_shape=[
            jax.ShapeDtypeStruct((B, FUSED), jnp.float32),
            jax.ShapeDtypeStruct((B, K), jnp.int32),
            jax.ShapeDtypeStruct((B, 16), jnp.float32),
            jax.ShapeDtypeStruct((1, 1), jnp.float32),
        ],
    )(x0, x1, x2, cc, ct, imr, imc,
      r(ln_g0), r(ln_b0), Wp0, r(bp0), r(ln_g1), r(ln_b1), Wp1, r(bp1),
      r(ln_g2), r(ln_b2), Wp2, r(bp2), Wg1, r(bg1), wg2p, bg2p)
    return fused, nidx, w16, ent


def _call_classifier(fused, agg, Wu1, bu1, Wu2, bu2, Wc1, bc1,
                     bn_g, bn_b, bn_m, bn_v, Wc2, bc2):
    r = lambda a: a.reshape(1, -1)
    row_spec = lambda c: pl.BlockSpec((BLK, c), lambda i: (i, 0))
    full_spec = lambda rr, c: pl.BlockSpec((rr, c), lambda i: (0, 0))
    logits = pl.pallas_call(
        _classifier_body,
        grid=(NBLK,),
        in_specs=[
            row_spec(FUSED), row_spec(FUSED),
            full_spec(FUSED, FUSED), full_spec(1, FUSED),
            full_spec(FUSED, FUSED), full_spec(1, FUSED),
            full_spec(FUSED, H), full_spec(1, H),
            full_spec(1, H), full_spec(1, H), full_spec(1, H), full_spec(1, H),
            full_spec(H, 5), full_spec(1, 5),
        ],
        out_specs=[row_spec(5)],
        out_shape=[jax.ShapeDtypeStruct((B, 5), jnp.float32)],
    )(fused, agg, Wu1, r(bu1), Wu2, r(bu2), Wc1, r(bc1),
      r(bn_g), r(bn_b), r(bn_m), r(bn_v), Wc2, r(bc2))[0]
    return logits


def kernel(x, ln_g0, ln_b0, Wp0, bp0, ln_g1, ln_b1, Wp1, bp1, ln_g2, ln_b2,
           Wp2, bp2, Wg1, bg1, Wg2, bg2, Wu1, bu1, Wu2, bu2, Wc1, bc1,
           bn_g, bn_b, bn_m, bn_v, Wc2, bc2):
    fused, nidx, w16, ent = _call_fuse_knn(
        x, ln_g0, ln_b0, Wp0, bp0, ln_g1, ln_b1, Wp1, bp1,
        ln_g2, ln_b2, Wp2, bp2, Wg1, bg1, Wg2, bg2)
    agg = _make_sc_agg()(fused, nidx.reshape(-1), w16)
    logits = _call_classifier(fused, agg, Wu1, bu1, Wu2, bu2, Wc1, bc1,
                              bn_g, bn_b, bn_m, bn_v, Wc2, bc2)
    return (logits, ent[0, 0])


# speedup vs baseline: 9.1833x; 1.2034x over previous
import functools

import jax
import jax.numpy as jnp
from jax import lax
from jax.experimental import pallas as pl
from jax.experimental.pallas import tpu as pltpu, tpu_sc as plsc

B = 4096
H = 256
FUSED = 768
K = 8
BLK = 256
NBLK = B // BLK
BIG = 1e30


def _ln(v, g, b):
    mu = jnp.mean(v, axis=-1, keepdims=True)
    var = jnp.mean((v - mu) ** 2, axis=-1, keepdims=True)
    return (v - mu) / jnp.sqrt(var + 1e-5) * g + b


def _fuse_knn_body(x0, x1, x2, cc, ct, imr, imc,
                   lg0, lb0, wp0, bp0, lg1, lb1, wp1, bp1, lg2, lb2, wp2, bp2,
                   wg1, bg1, wg2, bg2,
                   fused_o, nidx_o, w_o, ent_o):
    i = pl.program_id(0)
    v0 = jnp.dot(_ln(x0[...], lg0[...], lb0[...]), wp0[...],
                 preferred_element_type=jnp.float32) + bp0[...]
    v1 = jnp.dot(_ln(x1[...], lg1[...], lb1[...]), wp1[...],
                 preferred_element_type=jnp.float32) + bp1[...]
    v2 = jnp.dot(_ln(x2[...], lg2[...], lb2[...]), wp2[...],
                 preferred_element_type=jnp.float32) + bp2[...]
    concat = jnp.concatenate([v0, v1, v2], axis=1)
    gh = jnp.maximum(jnp.dot(concat, wg1[...], preferred_element_type=jnp.float32)
                     + bg1[...], 0.0)
    gl = jnp.dot(gh, wg2[...], preferred_element_type=jnp.float32) + bg2[...]
    gm = jnp.max(gl, axis=1, keepdims=True)
    ge = jnp.exp(gl - gm)
    gp = ge / jnp.sum(ge, axis=1, keepdims=True)
    ent_blk = -jnp.sum(gp * jnp.log(gp + 1e-8))

    @pl.when(i == 0)
    def _():
        ent_o[0, 0] = 0.0
    ent_o[0, 0] += ent_blk * (0.01 / B)

    fused_o[...] = jnp.concatenate(
        [v0 * gp[:, 0:1], v1 * gp[:, 1:2], v2 * gp[:, 2:3]], axis=1)

    cb = cc[...]
    cxb = cb[:, 0:1]
    cyb = cb[:, 1:2]
    cx = ct[0:1, :]
    cy = ct[1:2, :]
    sqr = cxb * cxb + cyb * cyb
    sqc = cx * cx + cy * cy
    bf = lambda a: a.astype(jnp.bfloat16).astype(jnp.float32)
    prod = bf(cxb) * bf(cx) + bf(cyb) * bf(cy)
    d2 = sqr + sqc - 2.0 * prod
    rowid = (jnp.float32(i * BLK) +
             lax.broadcasted_iota(jnp.int32, (BLK, 1), 0).astype(jnp.float32))
    colid = lax.broadcasted_iota(jnp.int32, (BLK, B), 1).astype(jnp.float32)
    diag = colid == rowid
    eq = imr[...] == imc[...]
    cnt = jnp.sum(jnp.where(eq, 1.0, 0.0), axis=1, keepdims=True)
    has_n = cnt > 1.5
    excl = diag | (has_n & jnp.logical_not(eq))
    sel = jnp.where(excl, BIG, d2)
    vals = []
    idxs = []
    for _ in range(K):
        mn = jnp.min(sel, axis=1, keepdims=True)
        mask = sel == mn
        amin = jnp.min(jnp.where(mask, colid, 1e9), axis=1, keepdims=True)
        vals.append(jnp.where(mn >= 1e29, -1e9,
                              -jnp.sqrt(jnp.maximum(mn, 1e-12))))
        idxs.append(amin)
        sel = jnp.where(colid == amin, BIG, sel)
    valsm = jnp.concatenate(vals, axis=1)
    nidx_o[...] = jnp.concatenate(idxs, axis=1).astype(jnp.int32)
    we = jnp.exp(valsm - valsm[:, 0:1])
    w = we / jnp.sum(we, axis=1, keepdims=True)
    w_o[...] = jnp.concatenate([w, jnp.zeros((BLK, 8), jnp.float32)], axis=1)


def _classifier_body(fused, agg, wu1, bu1, wu2, bu2, wc1, bc1,
                     bn_g, bn_b, bn_m, bn_v, wc2, bc2, out):
    h1 = jnp.maximum(jnp.dot(agg[...], wu1[...],
                             preferred_element_type=jnp.float32) + bu1[...], 0.0)
    upd = jnp.dot(h1, wu2[...], preferred_element_type=jnp.float32) + bu2[...]
    ctx = fused[...] + 0.5 * upd
    h = jnp.dot(ctx, wc1[...], preferred_element_type=jnp.float32) + bc1[...]
    h = (h - bn_m[...]) / jnp.sqrt(bn_v[...] + 1e-5) * bn_g[...] + bn_b[...]
    h = jnp.maximum(h, 0.0)
    out[...] = jnp.dot(h, wc2[...], preferred_element_type=jnp.float32) + bc2[...]


CH = 8
RPW = B // 32
NCH = RPW // CH


def _sc_agg_body(fused_hbm, idx_hbm, w_hbm, out_hbm,
                 idx_v, rows_v, w_v, agg_v, sem0, sem1):
    wid = lax.axis_index("s") * 2 + lax.axis_index("c")
    base = wid * RPW
    sems = (sem0, sem1)
    pltpu.sync_copy(w_hbm.at[pl.ds(base, RPW)], w_v)

    def start_gather(c, b):
        row0 = base + c * CH
        pltpu.sync_copy(idx_hbm.at[pl.ds(row0 * K, CH * K)], idx_v.at[b])
        pltpu.async_copy(fused_hbm.at[idx_v.at[b]], rows_v.at[b], sems[b])

    for b in range(2):
        start_gather(b, b)

    def compute_chunk(c, b):
        pltpu.make_async_copy(fused_hbm.at[idx_v.at[b]],
                              rows_v.at[b], sems[b]).wait()

        def row_body(rr, carry):
            wrow = w_v[c * CH + rr, :]
            splats = [wrow.at[jnp.full((16,), k, jnp.int32)]
                      .get(mode="promise_in_bounds") for k in range(K)]

            def s_body(s, carry3):
                col = s * 16
                a = splats[0] * rows_v[b, rr * K, pl.ds(col, 16)]
                for k in range(1, K):
                    a = a + splats[k] * rows_v[b, rr * K + k, pl.ds(col, 16)]
                agg_v[rr, pl.ds(col, 16)] = a
                return carry3

            return lax.fori_loop(0, FUSED // 16, s_body, carry)

        lax.fori_loop(0, CH, row_body, 0)
        pltpu.sync_copy(agg_v, out_hbm.at[pl.ds(base + c * CH, CH)])

    def superstep(g, carry):
        for b in range(2):
            c = 2 * g + b
            compute_chunk(c, b)

            @pl.when(c + 2 < NCH)
            def _():
                start_gather(c + 2, b)
        return carry

    lax.fori_loop(0, NCH // 2, superstep, 0)


def _make_sc_agg():
    mesh = plsc.VectorSubcoreMesh(core_axis_name="c", subcore_axis_name="s")
    return pl.kernel(
        _sc_agg_body,
        out_type=jax.ShapeDtypeStruct((B, FUSED), jnp.float32),
        mesh=mesh,
        scratch_types=[
            pltpu.VMEM((2, CH * K), jnp.int32),
            pltpu.VMEM((2, CH * K, FUSED), jnp.float32),
            pltpu.VMEM((RPW, 16), jnp.float32),
            pltpu.VMEM((CH, FUSED), jnp.float32),
            pltpu.SemaphoreType.DMA,
            pltpu.SemaphoreType.DMA,
        ],
    )


def _call_fuse_knn(x, ln_g0, ln_b0, Wp0, bp0, ln_g1, ln_b1, Wp1, bp1,
                   ln_g2, ln_b2, Wp2, bp2, Wg1, bg1, Wg2, bg2):
    x0 = x[:, 0:512]
    x1 = x[:, 512:1280]
    x2 = x[:, 1280:1536]
    cc = x[:, 1536:1538]
    ct = cc.T
    imc = x[:, 1540:1541]
    imr = imc.T
    r = lambda a: a.reshape(1, -1)
    wg2p = jnp.concatenate([Wg2, jnp.zeros((128, 5), jnp.float32)], axis=1)
    bg2p = jnp.concatenate([bg2, jnp.full((5,), -1e30, jnp.float32)]).reshape(1, 8)

    row_spec = lambda c: pl.BlockSpec((BLK, c), lambda i: (i, 0))
    full_spec = lambda rr, c: pl.BlockSpec((rr, c), lambda i: (0, 0))

    fused, nidx, w16, ent = pl.pallas_call(
        _fuse_knn_body,
        grid=(NBLK,),
        in_specs=[
            row_spec(512), row_spec(768), row_spec(256), row_spec(2),
            full_spec(2, B), full_spec(1, B), row_spec(1),
            full_spec(1, 512), full_spec(1, 512), full_spec(512, H), full_spec(1, H),
            full_spec(1, 768), full_spec(1, 768), full_spec(768, H), full_spec(1, H),
            full_spec(1, 256), full_spec(1, 256), full_spec(256, H), full_spec(1, H),
            full_spec(768, 128), full_spec(1, 128), full_spec(128, 8), full_spec(1, 8),
        ],
        out_specs=[
            row_spec(FUSED), row_spec(K), row_spec(16),
            pl.BlockSpec((1, 1), lambda i: (0, 0), memory_space=pltpu.SMEM),
        ],
        out_shape=[
            jax.ShapeDtypeStruct((B, FUSED), jnp.float32),
            jax.ShapeDtypeStruct((B, K), jnp.int32),
            jax.ShapeDtypeStruct((B, 16), jnp.float32),
            jax.ShapeDtypeStruct((1, 1), jnp.float32),
        ],
    )(x0, x1, x2, cc, ct, imr, imc,
      r(ln_g0), r(ln_b0), Wp0, r(bp0), r(ln_g1), r(ln_b1), Wp1, r(bp1),
      r(ln_g2), r(ln_b2), Wp2, r(bp2), Wg1, r(bg1), wg2p, bg2p)
    return fused, nidx, w16, ent


def _call_classifier(fused, agg, Wu1, bu1, Wu2, bu2, Wc1, bc1,
                     bn_g, bn_b, bn_m, bn_v, Wc2, bc2):
    r = lambda a: a.reshape(1, -1)
    row_spec = lambda c: pl.BlockSpec((BLK, c), lambda i: (i, 0))
    full_spec = lambda rr, c: pl.BlockSpec((rr, c), lambda i: (0, 0))
    logits = pl.pallas_call(
        _classifier_body,
        grid=(NBLK,),
        in_specs=[
            row_spec(FUSED), row_spec(FUSED),
            full_spec(FUSED, FUSED), full_spec(1, FUSED),
            full_spec(FUSED, FUSED), full_spec(1, FUSED),
            full_spec(FUSED, H), full_spec(1, H),
            full_spec(1, H), full_spec(1, H), full_spec(1, H), full_spec(1, H),
            full_spec(H, 5), full_spec(1, 5),
        ],
        out_specs=[row_spec(5)],
        out_shape=[jax.ShapeDtypeStruct((B, 5), jnp.float32)],
    )(fused, agg, Wu1, r(bu1), Wu2, r(bu2), Wc1, r(bc1),
      r(bn_g), r(bn_b), r(bn_m), r(bn_v), Wc2, r(bc2))[0]
    return logits


def kernel(x, ln_g0, ln_b0, Wp0, bp0, ln_g1, ln_b1, Wp1, bp1, ln_g2, ln_b2,
           Wp2, bp2, Wg1, bg1, Wg2, bg2, Wu1, bu1, Wu2, bu2, Wc1, bc1,
           bn_g, bn_b, bn_m, bn_v, Wc2, bc2):
    fused, nidx, w16, ent = _call_fuse_knn(
        x, ln_g0, ln_b0, Wp0, bp0, ln_g1, ln_b1, Wp1, bp1,
        ln_g2, ln_b2, Wp2, bp2, Wg1, bg1, Wg2, bg2)
    agg = _make_sc_agg()(fused, nidx.reshape(-1), w16)
    logits = _call_classifier(fused, agg, Wu1, bu1, Wu2, bu2, Wc1, bc1,
                              bn_g, bn_b, bn_m, bn_v, Wc2, bc2)
    return (logits, ent[0, 0])


# speedup vs baseline: 9.5538x; 1.0403x over previous
import functools

import jax
import jax.numpy as jnp
from jax import lax
from jax.experimental import pallas as pl
from jax.experimental.pallas import tpu as pltpu, tpu_sc as plsc

B = 4096
H = 256
FUSED = 768
K = 8
BLK = 256
NBLK = B // BLK
BIG = 1e30


def _ln(v, g, b):
    mu = jnp.mean(v, axis=-1, keepdims=True)
    var = jnp.mean((v - mu) ** 2, axis=-1, keepdims=True)
    return (v - mu) / jnp.sqrt(var + 1e-5) * g + b


def _fuse_knn_body(x0, x1a, x1b, x1c, x2, cc, ct, imr, imc,
                   lg0, lb0, wp0, bp0, lg1, lb1, wp1, bp1, lg2, lb2, wp2, bp2,
                   wg1, bg1, wg2, bg2,
                   fused_o, nidx_o, w_o, ent_o):
    i = pl.program_id(0)
    x1 = jnp.concatenate([x1a[...], x1b[...], x1c[...]], axis=1)
    v0 = jnp.dot(_ln(x0[...], lg0[...], lb0[...]), wp0[...],
                 preferred_element_type=jnp.float32) + bp0[...]
    v1 = jnp.dot(_ln(x1, lg1[...], lb1[...]), wp1[...],
                 preferred_element_type=jnp.float32) + bp1[...]
    v2 = jnp.dot(_ln(x2[...], lg2[...], lb2[...]), wp2[...],
                 preferred_element_type=jnp.float32) + bp2[...]
    concat = jnp.concatenate([v0, v1, v2], axis=1)
    gh = jnp.maximum(jnp.dot(concat, wg1[...], preferred_element_type=jnp.float32)
                     + bg1[...], 0.0)
    gl = jnp.dot(gh, wg2[...], preferred_element_type=jnp.float32) + bg2[...]
    gm = jnp.max(gl, axis=1, keepdims=True)
    ge = jnp.exp(gl - gm)
    gp = ge / jnp.sum(ge, axis=1, keepdims=True)
    ent_blk = -jnp.sum(gp * jnp.log(gp + 1e-8))

    @pl.when(i == 0)
    def _():
        ent_o[0, 0] = 0.0
    ent_o[0, 0] += ent_blk * (0.01 / B)

    fused_o[...] = jnp.concatenate(
        [v0 * gp[:, 0:1], v1 * gp[:, 1:2], v2 * gp[:, 2:3]], axis=1)

    cb = cc[...]
    cxb = cb[:, 0:1]
    cyb = cb[:, 1:2]
    cx = ct[0:1, :]
    cy = ct[1:2, :]
    sqr = cxb * cxb + cyb * cyb
    sqc = cx * cx + cy * cy
    bf = lambda a: a.astype(jnp.bfloat16).astype(jnp.float32)
    prod = bf(cxb) * bf(cx) + bf(cyb) * bf(cy)
    d2 = sqr + sqc - 2.0 * prod
    rowid = (jnp.float32(i * BLK) +
             lax.broadcasted_iota(jnp.int32, (BLK, 1), 0).astype(jnp.float32))
    colid = lax.broadcasted_iota(jnp.int32, (BLK, B), 1).astype(jnp.float32)
    diag = colid == rowid
    eq = imr[...] == imc[...]
    cnt = jnp.sum(jnp.where(eq, 1.0, 0.0), axis=1, keepdims=True)
    has_n = cnt > 1.5
    excl = diag | (has_n & jnp.logical_not(eq))
    sel = jnp.where(excl, BIG, d2)
    vals = []
    idxs = []
    for _ in range(K):
        mn = jnp.min(sel, axis=1, keepdims=True)
        mask = sel == mn
        amin = jnp.min(jnp.where(mask, colid, 1e9), axis=1, keepdims=True)
        vals.append(jnp.where(mn >= 1e29, -1e9,
                              -jnp.sqrt(jnp.maximum(mn, 1e-12))))
        idxs.append(amin)
        sel = jnp.where(colid == amin, BIG, sel)
    valsm = jnp.concatenate(vals, axis=1)
    nidx_o[...] = jnp.concatenate(idxs, axis=1).astype(jnp.int32)
    we = jnp.exp(valsm - valsm[:, 0:1])
    w = we / jnp.sum(we, axis=1, keepdims=True)
    w_o[...] = jnp.concatenate([w, jnp.zeros((BLK, 8), jnp.float32)], axis=1)


def _classifier_body(fused, agg, wu1, bu1, wu2, bu2, wc1, bc1,
                     bn_g, bn_b, bn_m, bn_v, wc2, bc2, out):
    h1 = jnp.maximum(jnp.dot(agg[...], wu1[...],
                             preferred_element_type=jnp.float32) + bu1[...], 0.0)
    upd = jnp.dot(h1, wu2[...], preferred_element_type=jnp.float32) + bu2[...]
    ctx = fused[...] + 0.5 * upd
    h = jnp.dot(ctx, wc1[...], preferred_element_type=jnp.float32) + bc1[...]
    h = (h - bn_m[...]) / jnp.sqrt(bn_v[...] + 1e-5) * bn_g[...] + bn_b[...]
    h = jnp.maximum(h, 0.0)
    out[...] = jnp.dot(h, wc2[...], preferred_element_type=jnp.float32) + bc2[...]


CH = 8
RPW = B // 32
NCH = RPW // CH


def _sc_agg_body(fused_hbm, idx_hbm, w_hbm, out_hbm,
                 idx_v, rows_v, w_v, agg_v, sem0, sem1):
    wid = lax.axis_index("s") * 2 + lax.axis_index("c")
    base = wid * RPW
    sems = (sem0, sem1)
    pltpu.sync_copy(w_hbm.at[pl.ds(base, RPW)], w_v)

    def start_gather(c, b):
        row0 = base + c * CH
        pltpu.sync_copy(idx_hbm.at[pl.ds(row0 * K, CH * K)], idx_v.at[b])
        pltpu.async_copy(fused_hbm.at[idx_v.at[b]], rows_v.at[b], sems[b])

    for b in range(2):
        start_gather(b, b)

    def compute_chunk(c, b):
        pltpu.make_async_copy(fused_hbm.at[idx_v.at[b]],
                              rows_v.at[b], sems[b]).wait()

        def row_body(rr, carry):
            wrow = w_v[c * CH + rr, :]
            splats = [wrow.at[jnp.full((16,), k, jnp.int32)]
                      .get(mode="promise_in_bounds") for k in range(K)]

            def s_body(s, carry3):
                for u in range(4):
                    col = s * 64 + u * 16
                    a = splats[0] * rows_v[b, rr * K, pl.ds(col, 16)]
                    for k in range(1, K):
                        a = a + splats[k] * rows_v[b, rr * K + k, pl.ds(col, 16)]
                    agg_v[rr, pl.ds(col, 16)] = a
                return carry3

            return lax.fori_loop(0, FUSED // 64, s_body, carry)

        lax.fori_loop(0, CH, row_body, 0)
        pltpu.sync_copy(agg_v, out_hbm.at[pl.ds(base + c * CH, CH)])

    def superstep(g, carry):
        for b in range(2):
            c = 2 * g + b
            compute_chunk(c, b)

            @pl.when(c + 2 < NCH)
            def _():
                start_gather(c + 2, b)
        return carry

    lax.fori_loop(0, NCH // 2, superstep, 0)


def _make_sc_agg():
    mesh = plsc.VectorSubcoreMesh(core_axis_name="c", subcore_axis_name="s")
    return pl.kernel(
        _sc_agg_body,
        out_type=jax.ShapeDtypeStruct((B, FUSED), jnp.float32),
        mesh=mesh,
        scratch_types=[
            pltpu.VMEM((2, CH * K), jnp.int32),
            pltpu.VMEM((2, CH * K, FUSED), jnp.float32),
            pltpu.VMEM((RPW, 16), jnp.float32),
            pltpu.VMEM((CH, FUSED), jnp.float32),
            pltpu.SemaphoreType.DMA,
            pltpu.SemaphoreType.DMA,
        ],
    )


def _call_fuse_knn(x, ln_g0, ln_b0, Wp0, bp0, ln_g1, ln_b1, Wp1, bp1,
                   ln_g2, ln_b2, Wp2, bp2, Wg1, bg1, Wg2, bg2):
    cc = x[:, 1536:1538]
    ct = cc.T
    imc = x[:, 1540:1541]
    imr = imc.T
    r = lambda a: a.reshape(1, -1)
    wg2p = jnp.concatenate([Wg2, jnp.zeros((128, 5), jnp.float32)], axis=1)
    bg2p = jnp.concatenate([bg2, jnp.full((5,), -1e30, jnp.float32)]).reshape(1, 8)

    row_spec = lambda c: pl.BlockSpec((BLK, c), lambda i: (i, 0))
    full_spec = lambda rr, c: pl.BlockSpec((rr, c), lambda i: (0, 0))

    fused, nidx, w16, ent = pl.pallas_call(
        _fuse_knn_body,
        grid=(NBLK,),
        in_specs=[
            pl.BlockSpec((BLK, 512), lambda i: (i, 0)),
            pl.BlockSpec((BLK, 256), lambda i: (i, 2)),
            pl.BlockSpec((BLK, 256), lambda i: (i, 3)),
            pl.BlockSpec((BLK, 256), lambda i: (i, 4)),
            pl.BlockSpec((BLK, 256), lambda i: (i, 5)),
            row_spec(2),
            full_spec(2, B), full_spec(1, B), row_spec(1),
            full_spec(1, 512), full_spec(1, 512), full_spec(512, H), full_spec(1, H),
            full_spec(1, 768), full_spec(1, 768), full_spec(768, H), full_spec(1, H),
            full_spec(1, 256), full_spec(1, 256), full_spec(256, H), full_spec(1, H),
            full_spec(768, 128), full_spec(1, 128), full_spec(128, 8), full_spec(1, 8),
        ],
        out_specs=[
            row_spec(FUSED), row_spec(K), row_spec(16),
            pl.BlockSpec((1, 1), lambda i: (0, 0), memory_space=pltpu.SMEM),
        ],
        out_shape=[
            jax.ShapeDtypeStruct((B, FUSED), jnp.float32),
            jax.ShapeDtypeStruct((B, K), jnp.int32),
            jax.ShapeDtypeStruct((B, 16), jnp.float32),
            jax.ShapeDtypeStruct((1, 1), jnp.float32),
        ],
    )(x, x, x, x, x, cc, ct, imr, imc,
      r(ln_g0), r(ln_b0), Wp0, r(bp0), r(ln_g1), r(ln_b1), Wp1, r(bp1),
      r(ln_g2), r(ln_b2), Wp2, r(bp2), Wg1, r(bg1), wg2p, bg2p)
    return fused, nidx, w16, ent


def _call_classifier(fused, agg, Wu1, bu1, Wu2, bu2, Wc1, bc1,
                     bn_g, bn_b, bn_m, bn_v, Wc2, bc2):
    r = lambda a: a.reshape(1, -1)
    row_spec = lambda c: pl.BlockSpec((BLK, c), lambda i: (i, 0))
    full_spec = lambda rr, c: pl.BlockSpec((rr, c), lambda i: (0, 0))
    logits = pl.pallas_call(
        _classifier_body,
        grid=(NBLK,),
        in_specs=[
            row_spec(FUSED), row_spec(FUSED),
            full_spec(FUSED, FUSED), full_spec(1, FUSED),
            full_spec(FUSED, FUSED), full_spec(1, FUSED),
            full_spec(FUSED, H), full_spec(1, H),
            full_spec(1, H), full_spec(1, H), full_spec(1, H), full_spec(1, H),
            full_spec(H, 5), full_spec(1, 5),
        ],
        out_specs=[row_spec(5)],
        out_shape=[jax.ShapeDtypeStruct((B, 5), jnp.float32)],
    )(fused, agg, Wu1, r(bu1), Wu2, r(bu2), Wc1, r(bc1),
      r(bn_g), r(bn_b), r(bn_m), r(bn_v), Wc2, r(bc2))[0]
    return logits


def kernel(x, ln_g0, ln_b0, Wp0, bp0, ln_g1, ln_b1, Wp1, bp1, ln_g2, ln_b2,
           Wp2, bp2, Wg1, bg1, Wg2, bg2, Wu1, bu1, Wu2, bu2, Wc1, bc1,
           bn_g, bn_b, bn_m, bn_v, Wc2, bc2):
    fused, nidx, w16, ent = _call_fuse_knn(
        x, ln_g0, ln_b0, Wp0, bp0, ln_g1, ln_b1, Wp1, bp1,
        ln_g2, ln_b2, Wp2, bp2, Wg1, bg1, Wg2, bg2)
    agg = _make_sc_agg()(fused, nidx.reshape(-1), w16)
    logits = _call_classifier(fused, agg, Wu1, bu1, Wu2, bu2, Wc1, bc1,
                              bn_g, bn_b, bn_m, bn_v, Wc2, bc2)
    return (logits, ent[0, 0])


# speedup vs baseline: 9.5556x; 1.0002x over previous
import functools

import jax
import jax.numpy as jnp
from jax import lax
from jax.experimental import pallas as pl
from jax.experimental.pallas import tpu as pltpu, tpu_sc as plsc

B = 4096
H = 256
FUSED = 768
K = 8
BLK = 256
NBLK = B // BLK
BIG = 1e30


def _ln(v, g, b):
    mu = jnp.mean(v, axis=-1, keepdims=True)
    var = jnp.mean((v - mu) ** 2, axis=-1, keepdims=True)
    return (v - mu) / jnp.sqrt(var + 1e-5) * g + b


def _fuse_knn_body(x0, x1a, x1b, x1c, x2, cc, ct, imr, imc,
                   lg0, lb0, wp0, bp0, lg1, lb1, wp1, bp1, lg2, lb2, wp2, bp2,
                   wg1, bg1, wg2, bg2,
                   fused_o, nidx_o, w_o, ent_o):
    i = pl.program_id(0)
    x1 = jnp.concatenate([x1a[...], x1b[...], x1c[...]], axis=1)
    v0 = jnp.dot(_ln(x0[...], lg0[...], lb0[...]), wp0[...],
                 preferred_element_type=jnp.float32) + bp0[...]
    v1 = jnp.dot(_ln(x1, lg1[...], lb1[...]), wp1[...],
                 preferred_element_type=jnp.float32) + bp1[...]
    v2 = jnp.dot(_ln(x2[...], lg2[...], lb2[...]), wp2[...],
                 preferred_element_type=jnp.float32) + bp2[...]
    concat = jnp.concatenate([v0, v1, v2], axis=1)
    gh = jnp.maximum(jnp.dot(concat, wg1[...], preferred_element_type=jnp.float32)
                     + bg1[...], 0.0)
    gl = jnp.dot(gh, wg2[...], preferred_element_type=jnp.float32) + bg2[...]
    gm = jnp.max(gl, axis=1, keepdims=True)
    ge = jnp.exp(gl - gm)
    gp = ge / jnp.sum(ge, axis=1, keepdims=True)
    ent_blk = -jnp.sum(gp * jnp.log(gp + 1e-8))

    @pl.when(i == 0)
    def _():
        ent_o[0, 0] = 0.0
    ent_o[0, 0] += ent_blk * (0.01 / B)

    fused_o[...] = jnp.concatenate(
        [v0 * gp[:, 0:1], v1 * gp[:, 1:2], v2 * gp[:, 2:3]], axis=1)

    cb = cc[...]
    cxb = cb[:, 0:1]
    cyb = cb[:, 1:2]
    cx = ct[0:1, :]
    cy = ct[1:2, :]
    sqr = cxb * cxb + cyb * cyb
    sqc = cx * cx + cy * cy
    bf = lambda a: a.astype(jnp.bfloat16).astype(jnp.float32)
    prod = bf(cxb) * bf(cx) + bf(cyb) * bf(cy)
    d2 = sqr + sqc - 2.0 * prod
    rowid = (jnp.float32(i * BLK) +
             lax.broadcasted_iota(jnp.int32, (BLK, 1), 0).astype(jnp.float32))
    colid = lax.broadcasted_iota(jnp.int32, (BLK, B), 1).astype(jnp.float32)
    diag = colid == rowid
    eq = imr[...] == imc[...]
    cnt = jnp.sum(jnp.where(eq, 1.0, 0.0), axis=1, keepdims=True)
    has_n = cnt > 1.5
    excl = diag | (has_n & jnp.logical_not(eq))
    sel = jnp.where(excl, BIG, d2)
    vals = []
    idxs = []
    for _ in range(K):
        mn = jnp.min(sel, axis=1, keepdims=True)
        mask = sel == mn
        amin = jnp.min(jnp.where(mask, colid, 1e9), axis=1, keepdims=True)
        vals.append(jnp.where(mn >= 1e29, -1e9,
                              -jnp.sqrt(jnp.maximum(mn, 1e-12))))
        idxs.append(amin)
        sel = jnp.where(colid == amin, BIG, sel)
    valsm = jnp.concatenate(vals, axis=1)
    nidx_o[...] = jnp.concatenate(idxs, axis=1).astype(jnp.int32)
    we = jnp.exp(valsm - valsm[:, 0:1])
    w = we / jnp.sum(we, axis=1, keepdims=True)
    w_o[...] = jnp.concatenate([w, jnp.zeros((BLK, 8), jnp.float32)], axis=1)


def _classifier_body(fused, agg, wu1, bu1, wu2, bu2, wc1, bc1,
                     bn_g, bn_b, bn_m, bn_v, wc2, bc2, out):
    h1 = jnp.maximum(jnp.dot(agg[...], wu1[...],
                             preferred_element_type=jnp.float32) + bu1[...], 0.0)
    upd = jnp.dot(h1, wu2[...], preferred_element_type=jnp.float32) + bu2[...]
    ctx = fused[...] + 0.5 * upd
    h = jnp.dot(ctx, wc1[...], preferred_element_type=jnp.float32) + bc1[...]
    h = (h - bn_m[...]) / jnp.sqrt(bn_v[...] + 1e-5) * bn_g[...] + bn_b[...]
    h = jnp.maximum(h, 0.0)
    out[...] = jnp.dot(h, wc2[...], preferred_element_type=jnp.float32) + bc2[...]


CH = 8
RPW = B // 32
NCH = RPW // CH
PKW = FUSED // 2


def _sc_agg_body(fused_hbm, idx_hbm, w_hbm, out_hbm,
                 idx_v, rows_v, w_v, agg_v, sem0, sem1):
    wid = lax.axis_index("s") * 2 + lax.axis_index("c")
    base = wid * RPW
    sems = (sem0, sem1)
    pltpu.sync_copy(w_hbm.at[pl.ds(base, RPW)], w_v)

    def start_gather(c, b):
        row0 = base + c * CH
        pltpu.sync_copy(idx_hbm.at[pl.ds(row0 * K, CH * K)], idx_v.at[b])
        pltpu.async_copy(fused_hbm.at[idx_v.at[b]], rows_v.at[b], sems[b])

    for b in range(2):
        start_gather(b, b)

    def compute_chunk(c, b):
        pltpu.make_async_copy(fused_hbm.at[idx_v.at[b]],
                              rows_v.at[b], sems[b]).wait()

        def row_body(rr, carry):
            wrow = w_v[c * CH + rr, :]
            splats = [wrow.at[jnp.full((16,), k, jnp.int32)]
                      .get(mode="promise_in_bounds") for k in range(K)]

            def s_body(s, carry3):
                for u in range(4):
                    col = s * 64 + u * 16
                    a = splats[0] * rows_v[b, rr * K, pl.ds(col, 16)]
                    for k in range(1, K):
                        a = a + splats[k] * rows_v[b, rr * K + k, pl.ds(col, 16)]
                    agg_v[rr, pl.ds(col, 16)] = a
                return carry3

            return lax.fori_loop(0, FUSED // 64, s_body, carry)

        lax.fori_loop(0, CH, row_body, 0)
        pltpu.sync_copy(agg_v, out_hbm.at[pl.ds(base + c * CH, CH)])

    def superstep(g, carry):
        for b in range(2):
            c = 2 * g + b
            compute_chunk(c, b)

            @pl.when(c + 2 < NCH)
            def _():
                start_gather(c + 2, b)
        return carry

    lax.fori_loop(0, NCH // 2, superstep, 0)


def _make_sc_agg():
    mesh = plsc.VectorSubcoreMesh(core_axis_name="c", subcore_axis_name="s")
    return pl.kernel(
        _sc_agg_body,
        out_type=jax.ShapeDtypeStruct((B, FUSED), jnp.float32),
        mesh=mesh,
        scratch_types=[
            pltpu.VMEM((2, CH * K), jnp.int32),
            pltpu.VMEM((2, CH * K, FUSED), jnp.float32),
            pltpu.VMEM((RPW, 16), jnp.float32),
            pltpu.VMEM((CH, FUSED), jnp.float32),
            pltpu.SemaphoreType.DMA,
            pltpu.SemaphoreType.DMA,
        ],
    )


def _call_fuse_knn(x, ln_g0, ln_b0, Wp0, bp0, ln_g1, ln_b1, Wp1, bp1,
                   ln_g2, ln_b2, Wp2, bp2, Wg1, bg1, Wg2, bg2):
    cc = x[:, 1536:1538]
    ct = cc.T
    imc = x[:, 1540:1541]
    imr = imc.T
    r = lambda a: a.reshape(1, -1)
    wg2p = jnp.concatenate([Wg2, jnp.zeros((128, 5), jnp.float32)], axis=1)
    bg2p = jnp.concatenate([bg2, jnp.full((5,), -1e30, jnp.float32)]).reshape(1, 8)

    row_spec = lambda c: pl.BlockSpec((BLK, c), lambda i: (i, 0))
    full_spec = lambda rr, c: pl.BlockSpec((rr, c), lambda i: (0, 0))

    fused, nidx, w16, ent = pl.pallas_call(
        _fuse_knn_body,
        grid=(NBLK,),
        in_specs=[
            pl.BlockSpec((BLK, 512), lambda i: (i, 0)),
            pl.BlockSpec((BLK, 256), lambda i: (i, 2)),
            pl.BlockSpec((BLK, 256), lambda i: (i, 3)),
            pl.BlockSpec((BLK, 256), lambda i: (i, 4)),
            pl.BlockSpec((BLK, 256), lambda i: (i, 5)),
            row_spec(2),
            full_spec(2, B), full_spec(1, B), row_spec(1),
            full_spec(1, 512), full_spec(1, 512), full_spec(512, H), full_spec(1, H),
            full_spec(1, 768), full_spec(1, 768), full_spec(768, H), full_spec(1, H),
            full_spec(1, 256), full_spec(1, 256), full_spec(256, H), full_spec(1, H),
            full_spec(768, 128), full_spec(1, 128), full_spec(128, 8), full_spec(1, 8),
        ],
        out_specs=[
            row_spec(FUSED), row_spec(K), row_spec(16),
            pl.BlockSpec((1, 1), lambda i: (0, 0), memory_space=pltpu.SMEM),
        ],
        out_shape=[
            jax.ShapeDtypeStruct((B, FUSED), jnp.float32),
            jax.ShapeDtypeStruct((B, K), jnp.int32),
            jax.ShapeDtypeStruct((B, 16), jnp.float32),
            jax.ShapeDtypeStruct((1, 1), jnp.float32),
        ],
    )(x, x, x, x, x, cc, ct, imr, imc,
      r(ln_g0), r(ln_b0), Wp0, r(bp0), r(ln_g1), r(ln_b1), Wp1, r(bp1),
      r(ln_g2), r(ln_b2), Wp2, r(bp2), Wg1, r(bg1), wg2p, bg2p)
    return fused, nidx, w16, ent


def _call_classifier(fused, agg, Wu1, bu1, Wu2, bu2, Wc1, bc1,
                     bn_g, bn_b, bn_m, bn_v, Wc2, bc2):
    r = lambda a: a.reshape(1, -1)
    row_spec = lambda c: pl.BlockSpec((BLK, c), lambda i: (i, 0))
    full_spec = lambda rr, c: pl.BlockSpec((rr, c), lambda i: (0, 0))
    logits = pl.pallas_call(
        _classifier_body,
        grid=(NBLK,),
        in_specs=[
            row_spec(FUSED), row_spec(FUSED),
            full_spec(FUSED, FUSED), full_spec(1, FUSED),
            full_spec(FUSED, FUSED), full_spec(1, FUSED),
            full_spec(FUSED, H), full_spec(1, H),
            full_spec(1, H), full_spec(1, H), full_spec(1, H), full_spec(1, H),
            full_spec(H, 5), full_spec(1, 5),
        ],
        out_specs=[row_spec(5)],
        out_shape=[jax.ShapeDtypeStruct((B, 5), jnp.float32)],
    )(fused, agg, Wu1, r(bu1), Wu2, r(bu2), Wc1, r(bc1),
      r(bn_g), r(bn_b), r(bn_m), r(bn_v), Wc2, r(bc2))[0]
    return logits


def kernel(x, ln_g0, ln_b0, Wp0, bp0, ln_g1, ln_b1, Wp1, bp1, ln_g2, ln_b2,
           Wp2, bp2, Wg1, bg1, Wg2, bg2, Wu1, bu1, Wu2, bu2, Wc1, bc1,
           bn_g, bn_b, bn_m, bn_v, Wc2, bc2):
    fused, nidx, w16, ent = _call_fuse_knn(
        x, ln_g0, ln_b0, Wp0, bp0, ln_g1, ln_b1, Wp1, bp1,
        ln_g2, ln_b2, Wp2, bp2, Wg1, bg1, Wg2, bg2)
    agg = _make_sc_agg()(fused, nidx.reshape(-1), w16)
    logits = _call_classifier(fused, agg, Wu1, bu1, Wu2, bu2, Wc1, bc1,
                              bn_g, bn_b, bn_m, bn_v, Wc2, bc2)
    return (logits, ent[0, 0])


# speedup vs baseline: 9.6641x; 1.0114x over previous
import functools

import jax
import jax.numpy as jnp
from jax import lax
from jax.experimental import pallas as pl
from jax.experimental.pallas import tpu as pltpu, tpu_sc as plsc

B = 4096
H = 256
FUSED = 768
K = 8
BLK = 512
NBLK = B // BLK
BIG = 1e30


def _ln(v, g, b):
    mu = jnp.mean(v, axis=-1, keepdims=True)
    var = jnp.mean((v - mu) ** 2, axis=-1, keepdims=True)
    return (v - mu) / jnp.sqrt(var + 1e-5) * g + b


def _fuse_knn_body(x0, x1a, x1b, x1c, x2, cc, ct, imr, imc,
                   lg0, lb0, wp0, bp0, lg1, lb1, wp1, bp1, lg2, lb2, wp2, bp2,
                   wg1, bg1, wg2, bg2,
                   fused_o, nidx_o, w_o, ent_o):
    i = pl.program_id(0)
    x1 = jnp.concatenate([x1a[...], x1b[...], x1c[...]], axis=1)
    v0 = jnp.dot(_ln(x0[...], lg0[...], lb0[...]), wp0[...],
                 preferred_element_type=jnp.float32) + bp0[...]
    v1 = jnp.dot(_ln(x1, lg1[...], lb1[...]), wp1[...],
                 preferred_element_type=jnp.float32) + bp1[...]
    v2 = jnp.dot(_ln(x2[...], lg2[...], lb2[...]), wp2[...],
                 preferred_element_type=jnp.float32) + bp2[...]
    concat = jnp.concatenate([v0, v1, v2], axis=1)
    gh = jnp.maximum(jnp.dot(concat, wg1[...], preferred_element_type=jnp.float32)
                     + bg1[...], 0.0)
    gl = jnp.dot(gh, wg2[...], preferred_element_type=jnp.float32) + bg2[...]
    gm = jnp.max(gl, axis=1, keepdims=True)
    ge = jnp.exp(gl - gm)
    gp = ge / jnp.sum(ge, axis=1, keepdims=True)
    ent_blk = -jnp.sum(gp * jnp.log(gp + 1e-8))

    @pl.when(i == 0)
    def _():
        ent_o[0, 0] = 0.0
    ent_o[0, 0] += ent_blk * (0.01 / B)

    fused_o[...] = jnp.concatenate(
        [v0 * gp[:, 0:1], v1 * gp[:, 1:2], v2 * gp[:, 2:3]], axis=1)

    cb = cc[...]
    cxb = cb[:, 0:1]
    cyb = cb[:, 1:2]
    cx = ct[0:1, :]
    cy = ct[1:2, :]
    sqr = cxb * cxb + cyb * cyb
    sqc = cx * cx + cy * cy
    bf = lambda a: a.astype(jnp.bfloat16).astype(jnp.float32)
    prod = bf(cxb) * bf(cx) + bf(cyb) * bf(cy)
    d2 = sqr + sqc - 2.0 * prod
    rowid = (jnp.float32(i * BLK) +
             lax.broadcasted_iota(jnp.int32, (BLK, 1), 0).astype(jnp.float32))
    colid = lax.broadcasted_iota(jnp.int32, (BLK, B), 1).astype(jnp.float32)
    diag = colid == rowid
    eq = imr[...] == imc[...]
    cnt = jnp.sum(jnp.where(eq, 1.0, 0.0), axis=1, keepdims=True)
    has_n = cnt > 1.5
    excl = diag | (has_n & jnp.logical_not(eq))
    sel = jnp.where(excl, BIG, d2)
    vals = []
    idxs = []
    for _ in range(K):
        mn = jnp.min(sel, axis=1, keepdims=True)
        mask = sel == mn
        amin = jnp.min(jnp.where(mask, colid, 1e9), axis=1, keepdims=True)
        vals.append(jnp.where(mn >= 1e29, -1e9,
                              -jnp.sqrt(jnp.maximum(mn, 1e-12))))
        idxs.append(amin)
        sel = jnp.where(colid == amin, BIG, sel)
    valsm = jnp.concatenate(vals, axis=1)
    nidx_o[...] = jnp.concatenate(idxs, axis=1).astype(jnp.int32)
    we = jnp.exp(valsm - valsm[:, 0:1])
    w = we / jnp.sum(we, axis=1, keepdims=True)
    w_o[...] = jnp.concatenate([w, jnp.zeros((BLK, 8), jnp.float32)], axis=1)


def _classifier_body(fused, agg, wu1, bu1, wu2, bu2, wc1, bc1,
                     bn_g, bn_b, bn_m, bn_v, wc2, bc2, out):
    h1 = jnp.maximum(jnp.dot(agg[...], wu1[...],
                             preferred_element_type=jnp.float32) + bu1[...], 0.0)
    upd = jnp.dot(h1, wu2[...], preferred_element_type=jnp.float32) + bu2[...]
    ctx = fused[...] + 0.5 * upd
    h = jnp.dot(ctx, wc1[...], preferred_element_type=jnp.float32) + bc1[...]
    h = (h - bn_m[...]) / jnp.sqrt(bn_v[...] + 1e-5) * bn_g[...] + bn_b[...]
    h = jnp.maximum(h, 0.0)
    out[...] = jnp.dot(h, wc2[...], preferred_element_type=jnp.float32) + bc2[...]


CH = 8
RPW = B // 32
NCH = RPW // CH
PKW = FUSED // 2


def _sc_agg_body(fused_hbm, idx_hbm, w_hbm, out_hbm,
                 idx_v, rows_v, w_v, agg_v, sem0, sem1):
    wid = lax.axis_index("s") * 2 + lax.axis_index("c")
    base = wid * RPW
    sems = (sem0, sem1)
    pltpu.sync_copy(w_hbm.at[pl.ds(base, RPW)], w_v)

    def start_gather(c, b):
        row0 = base + c * CH
        pltpu.sync_copy(idx_hbm.at[pl.ds(row0 * K, CH * K)], idx_v.at[b])
        pltpu.async_copy(fused_hbm.at[idx_v.at[b]], rows_v.at[b], sems[b])

    for b in range(2):
        start_gather(b, b)

    def compute_chunk(c, b):
        pltpu.make_async_copy(fused_hbm.at[idx_v.at[b]],
                              rows_v.at[b], sems[b]).wait()

        def row_body(rr, carry):
            wrow = w_v[c * CH + rr, :]
            splats = [wrow.at[jnp.full((16,), k, jnp.int32)]
                      .get(mode="promise_in_bounds") for k in range(K)]

            def s_body(s, carry3):
                for u in range(4):
                    col = s * 64 + u * 16
                    a = splats[0] * rows_v[b, rr * K, pl.ds(col, 16)]
                    for k in range(1, K):
                        a = a + splats[k] * rows_v[b, rr * K + k, pl.ds(col, 16)]
                    agg_v[rr, pl.ds(col, 16)] = a
                return carry3

            return lax.fori_loop(0, FUSED // 64, s_body, carry)

        lax.fori_loop(0, CH, row_body, 0)
        pltpu.sync_copy(agg_v, out_hbm.at[pl.ds(base + c * CH, CH)])

    def superstep(g, carry):
        for b in range(2):
            c = 2 * g + b
            compute_chunk(c, b)

            @pl.when(c + 2 < NCH)
            def _():
                start_gather(c + 2, b)
        return carry

    lax.fori_loop(0, NCH // 2, superstep, 0)


def _make_sc_agg():
    mesh = plsc.VectorSubcoreMesh(core_axis_name="c", subcore_axis_name="s")
    return pl.kernel(
        _sc_agg_body,
        out_type=jax.ShapeDtypeStruct((B, FUSED), jnp.float32),
        mesh=mesh,
        scratch_types=[
            pltpu.VMEM((2, CH * K), jnp.int32),
            pltpu.VMEM((2, CH * K, FUSED), jnp.float32),
            pltpu.VMEM((RPW, 16), jnp.float32),
            pltpu.VMEM((CH, FUSED), jnp.float32),
            pltpu.SemaphoreType.DMA,
            pltpu.SemaphoreType.DMA,
        ],
    )


def _call_fuse_knn(x, ln_g0, ln_b0, Wp0, bp0, ln_g1, ln_b1, Wp1, bp1,
                   ln_g2, ln_b2, Wp2, bp2, Wg1, bg1, Wg2, bg2):
    cc = x[:, 1536:1538]
    ct = cc.T
    imc = x[:, 1540:1541]
    imr = imc.T
    r = lambda a: a.reshape(1, -1)
    wg2p = jnp.concatenate([Wg2, jnp.zeros((128, 5), jnp.float32)], axis=1)
    bg2p = jnp.concatenate([bg2, jnp.full((5,), -1e30, jnp.float32)]).reshape(1, 8)

    row_spec = lambda c: pl.BlockSpec((BLK, c), lambda i: (i, 0))
    full_spec = lambda rr, c: pl.BlockSpec((rr, c), lambda i: (0, 0))

    fused, nidx, w16, ent = pl.pallas_call(
        _fuse_knn_body,
        grid=(NBLK,),
        in_specs=[
            pl.BlockSpec((BLK, 512), lambda i: (i, 0)),
            pl.BlockSpec((BLK, 256), lambda i: (i, 2)),
            pl.BlockSpec((BLK, 256), lambda i: (i, 3)),
            pl.BlockSpec((BLK, 256), lambda i: (i, 4)),
            pl.BlockSpec((BLK, 256), lambda i: (i, 5)),
            row_spec(2),
            full_spec(2, B), full_spec(1, B), row_spec(1),
            full_spec(1, 512), full_spec(1, 512), full_spec(512, H), full_spec(1, H),
            full_spec(1, 768), full_spec(1, 768), full_spec(768, H), full_spec(1, H),
            full_spec(1, 256), full_spec(1, 256), full_spec(256, H), full_spec(1, H),
            full_spec(768, 128), full_spec(1, 128), full_spec(128, 8), full_spec(1, 8),
        ],
        out_specs=[
            row_spec(FUSED), row_spec(K), row_spec(16),
            pl.BlockSpec((1, 1), lambda i: (0, 0), memory_space=pltpu.SMEM),
        ],
        out_shape=[
            jax.ShapeDtypeStruct((B, FUSED), jnp.float32),
            jax.ShapeDtypeStruct((B, K), jnp.int32),
            jax.ShapeDtypeStruct((B, 16), jnp.float32),
            jax.ShapeDtypeStruct((1, 1), jnp.float32),
        ],
    )(x, x, x, x, x, cc, ct, imr, imc,
      r(ln_g0), r(ln_b0), Wp0, r(bp0), r(ln_g1), r(ln_b1), Wp1, r(bp1),
      r(ln_g2), r(ln_b2), Wp2, r(bp2), Wg1, r(bg1), wg2p, bg2p)
    return fused, nidx, w16, ent


def _call_classifier(fused, agg, Wu1, bu1, Wu2, bu2, Wc1, bc1,
                     bn_g, bn_b, bn_m, bn_v, Wc2, bc2):
    r = lambda a: a.reshape(1, -1)
    row_spec = lambda c: pl.BlockSpec((BLK, c), lambda i: (i, 0))
    full_spec = lambda rr, c: pl.BlockSpec((rr, c), lambda i: (0, 0))
    logits = pl.pallas_call(
        _classifier_body,
        grid=(NBLK,),
        in_specs=[
            row_spec(FUSED), row_spec(FUSED),
            full_spec(FUSED, FUSED), full_spec(1, FUSED),
            full_spec(FUSED, FUSED), full_spec(1, FUSED),
            full_spec(FUSED, H), full_spec(1, H),
            full_spec(1, H), full_spec(1, H), full_spec(1, H), full_spec(1, H),
            full_spec(H, 5), full_spec(1, 5),
        ],
        out_specs=[row_spec(5)],
        out_shape=[jax.ShapeDtypeStruct((B, 5), jnp.float32)],
    )(fused, agg, Wu1, r(bu1), Wu2, r(bu2), Wc1, r(bc1),
      r(bn_g), r(bn_b), r(bn_m), r(bn_v), Wc2, r(bc2))[0]
    return logits


def kernel(x, ln_g0, ln_b0, Wp0, bp0, ln_g1, ln_b1, Wp1, bp1, ln_g2, ln_b2,
           Wp2, bp2, Wg1, bg1, Wg2, bg2, Wu1, bu1, Wu2, bu2, Wc1, bc1,
           bn_g, bn_b, bn_m, bn_v, Wc2, bc2):
    fused, nidx, w16, ent = _call_fuse_knn(
        x, ln_g0, ln_b0, Wp0, bp0, ln_g1, ln_b1, Wp1, bp1,
        ln_g2, ln_b2, Wp2, bp2, Wg1, bg1, Wg2, bg2)
    agg = _make_sc_agg()(fused, nidx.reshape(-1), w16)
    logits = _call_classifier(fused, agg, Wu1, bu1, Wu2, bu2, Wc1, bc1,
                              bn_g, bn_b, bn_m, bn_v, Wc2, bc2)
    return (logits, ent[0, 0])


# speedup vs baseline: 9.6735x; 1.0010x over previous
import jax
import jax.numpy as jnp
from jax import lax
from jax.experimental import pallas as pl
from jax.experimental.pallas import tpu as pltpu, tpu_sc as plsc

B = 4096
H = 256
FUSED = 768
K = 8
BLK = 512
NBLK = B // BLK
BIG = 1e30


def _ln(v, g, b):
    mu = jnp.mean(v, axis=-1, keepdims=True)
    var = jnp.mean((v - mu) ** 2, axis=-1, keepdims=True)
    return (v - mu) / jnp.sqrt(var + 1e-5) * g + b


def _fuse_knn_body(x0, x1a, x1b, x1c, x2, cc, ct, imr, imc,
                   lg0, lb0, wp0, bp0, lg1, lb1, wp1, bp1, lg2, lb2, wp2, bp2,
                   wg1, bg1, wg2, bg2,
                   fused_o, nidx_o, w_o, ent_o):
    i = pl.program_id(0)
    x1 = jnp.concatenate([x1a[...], x1b[...], x1c[...]], axis=1)
    v0 = jnp.dot(_ln(x0[...], lg0[...], lb0[...]), wp0[...],
                 preferred_element_type=jnp.float32) + bp0[...]
    v1 = jnp.dot(_ln(x1, lg1[...], lb1[...]), wp1[...],
                 preferred_element_type=jnp.float32) + bp1[...]
    v2 = jnp.dot(_ln(x2[...], lg2[...], lb2[...]), wp2[...],
                 preferred_element_type=jnp.float32) + bp2[...]
    concat = jnp.concatenate([v0, v1, v2], axis=1)
    gh = jnp.maximum(jnp.dot(concat, wg1[...], preferred_element_type=jnp.float32)
                     + bg1[...], 0.0)
    gl = jnp.dot(gh, wg2[...], preferred_element_type=jnp.float32) + bg2[...]
    gm = jnp.max(gl, axis=1, keepdims=True)
    ge = jnp.exp(gl - gm)
    gp = ge / jnp.sum(ge, axis=1, keepdims=True)
    ent_blk = -jnp.sum(gp * jnp.log(gp + 1e-8))

    @pl.when(i == 0)
    def _():
        ent_o[0, 0] = 0.0
    ent_o[0, 0] += ent_blk * (0.01 / B)

    fused_o[...] = jnp.concatenate(
        [v0 * gp[:, 0:1], v1 * gp[:, 1:2], v2 * gp[:, 2:3]], axis=1)

    cb = cc[...]
    cxb = cb[:, 0:1]
    cyb = cb[:, 1:2]
    cx = ct[0:1, :]
    cy = ct[1:2, :]
    sqr = cxb * cxb + cyb * cyb
    sqc = cx * cx + cy * cy
    bf = lambda a: a.astype(jnp.bfloat16).astype(jnp.float32)
    prod = bf(cxb) * bf(cx) + bf(cyb) * bf(cy)
    d2 = sqr + sqc - 2.0 * prod
    rowid = (jnp.float32(i * BLK) +
             lax.broadcasted_iota(jnp.int32, (BLK, 1), 0).astype(jnp.float32))
    colid = lax.broadcasted_iota(jnp.int32, (BLK, B), 1).astype(jnp.float32)
    diag = colid == rowid
    eq = imr[...] == imc[...]
    cnt = jnp.sum(jnp.where(eq, 1.0, 0.0), axis=1, keepdims=True)
    has_n = cnt > 1.5
    excl = diag | (has_n & jnp.logical_not(eq))
    sel = jnp.where(excl, BIG, d2)
    vals = []
    idxs = []
    for _ in range(K):
        mn = jnp.min(sel, axis=1, keepdims=True)
        mask = sel == mn
        amin = jnp.min(jnp.where(mask, colid, 1e9), axis=1, keepdims=True)
        vals.append(jnp.where(mn >= 1e29, -1e9,
                              -jnp.sqrt(jnp.maximum(mn, 1e-12))))
        idxs.append(amin)
        sel = jnp.where(colid == amin, BIG, sel)
    valsm = jnp.concatenate(vals, axis=1)
    nidx_o[...] = jnp.concatenate(idxs, axis=1).astype(jnp.int32)
    we = jnp.exp(valsm - valsm[:, 0:1])
    w = we / jnp.sum(we, axis=1, keepdims=True)
    w_o[...] = jnp.concatenate([w, jnp.zeros((BLK, 8), jnp.float32)], axis=1)


def _classifier_body(fused, agg, wu1, bu1, wu2, bu2, wc1, bc1,
                     bn_g, bn_b, bn_m, bn_v, wc2, bc2, out):
    h1 = jnp.maximum(jnp.dot(agg[...], wu1[...],
                             preferred_element_type=jnp.float32) + bu1[...], 0.0)
    upd = jnp.dot(h1, wu2[...], preferred_element_type=jnp.float32) + bu2[...]
    ctx = fused[...] + 0.5 * upd
    h = jnp.dot(ctx, wc1[...], preferred_element_type=jnp.float32) + bc1[...]
    h = (h - bn_m[...]) / jnp.sqrt(bn_v[...] + 1e-5) * bn_g[...] + bn_b[...]
    h = jnp.maximum(h, 0.0)
    out[...] = jnp.dot(h, wc2[...], preferred_element_type=jnp.float32) + bc2[...]


CH = 8
RPW = B // 32
NCH = RPW // CH


def _sc_agg_body(fused_hbm, idx_hbm, w_hbm, out_hbm,
                 idx_v, rows_v, w_v, agg_v, sem0, sem1):
    wid = lax.axis_index("s") * 2 + lax.axis_index("c")
    base = wid * RPW
    sems = (sem0, sem1)
    pltpu.sync_copy(w_hbm.at[pl.ds(base, RPW)], w_v)

    def start_gather(c, b):
        row0 = base + c * CH
        pltpu.sync_copy(idx_hbm.at[pl.ds(row0 * K, CH * K)], idx_v.at[b])
        pltpu.async_copy(fused_hbm.at[idx_v.at[b]], rows_v.at[b], sems[b])

    for b in range(2):
        start_gather(b, b)

    def compute_chunk(c, b):
        pltpu.make_async_copy(fused_hbm.at[idx_v.at[b]],
                              rows_v.at[b], sems[b]).wait()

        def row_body(rr, carry):
            wrow = w_v[c * CH + rr, :]
            splats = [wrow.at[jnp.full((16,), k, jnp.int32)]
                      .get(mode="promise_in_bounds") for k in range(K)]

            def s_body(s, carry3):
                for u in range(4):
                    col = s * 64 + u * 16
                    a = splats[0] * rows_v[b, rr * K, pl.ds(col, 16)]
                    for k in range(1, K):
                        a = a + splats[k] * rows_v[b, rr * K + k, pl.ds(col, 16)]
                    agg_v[rr, pl.ds(col, 16)] = a
                return carry3

            return lax.fori_loop(0, FUSED // 64, s_body, carry)

        lax.fori_loop(0, CH, row_body, 0)
        pltpu.sync_copy(agg_v, out_hbm.at[pl.ds(base + c * CH, CH)])

    def superstep(g, carry):
        for b in range(2):
            c = 2 * g + b
            compute_chunk(c, b)

            @pl.when(c + 2 < NCH)
            def _():
                start_gather(c + 2, b)
        return carry

    lax.fori_loop(0, NCH // 2, superstep, 0)


def _make_sc_agg():
    mesh = plsc.VectorSubcoreMesh(core_axis_name="c", subcore_axis_name="s")
    return pl.kernel(
        _sc_agg_body,
        out_type=jax.ShapeDtypeStruct((B, FUSED), jnp.float32),
        mesh=mesh,
        scratch_types=[
            pltpu.VMEM((2, CH * K), jnp.int32),
            pltpu.VMEM((2, CH * K, FUSED), jnp.float32),
            pltpu.VMEM((RPW, 16), jnp.float32),
            pltpu.VMEM((CH, FUSED), jnp.float32),
            pltpu.SemaphoreType.DMA,
            pltpu.SemaphoreType.DMA,
        ],
    )


def _call_fuse_knn(x, ln_g0, ln_b0, Wp0, bp0, ln_g1, ln_b1, Wp1, bp1,
                   ln_g2, ln_b2, Wp2, bp2, Wg1, bg1, Wg2, bg2):
    cc = x[:, 1536:1538]
    ct = cc.T
    imc = x[:, 1540:1541]
    imr = imc.T
    r = lambda a: a.reshape(1, -1)
    wg2p = jnp.concatenate([Wg2, jnp.zeros((128, 5), jnp.float32)], axis=1)
    bg2p = jnp.concatenate([bg2, jnp.full((5,), -1e30, jnp.float32)]).reshape(1, 8)

    row_spec = lambda c: pl.BlockSpec((BLK, c), lambda i: (i, 0))
    full_spec = lambda rr, c: pl.BlockSpec((rr, c), lambda i: (0, 0))

    fused, nidx, w16, ent = pl.pallas_call(
        _fuse_knn_body,
        grid=(NBLK,),
        in_specs=[
            pl.BlockSpec((BLK, 512), lambda i: (i, 0)),
            pl.BlockSpec((BLK, 256), lambda i: (i, 2)),
            pl.BlockSpec((BLK, 256), lambda i: (i, 3)),
            pl.BlockSpec((BLK, 256), lambda i: (i, 4)),
            pl.BlockSpec((BLK, 256), lambda i: (i, 5)),
            row_spec(2),
            full_spec(2, B), full_spec(1, B), row_spec(1),
            full_spec(1, 512), full_spec(1, 512), full_spec(512, H), full_spec(1, H),
            full_spec(1, 768), full_spec(1, 768), full_spec(768, H), full_spec(1, H),
            full_spec(1, 256), full_spec(1, 256), full_spec(256, H), full_spec(1, H),
            full_spec(768, 128), full_spec(1, 128), full_spec(128, 8), full_spec(1, 8),
        ],
        out_specs=[
            row_spec(FUSED), row_spec(K), row_spec(16),
            pl.BlockSpec((1, 1), lambda i: (0, 0), memory_space=pltpu.SMEM),
        ],
        out_shape=[
            jax.ShapeDtypeStruct((B, FUSED), jnp.float32),
            jax.ShapeDtypeStruct((B, K), jnp.int32),
            jax.ShapeDtypeStruct((B, 16), jnp.float32),
            jax.ShapeDtypeStruct((1, 1), jnp.float32),
        ],
    )(x, x, x, x, x, cc, ct, imr, imc,
      r(ln_g0), r(ln_b0), Wp0, r(bp0), r(ln_g1), r(ln_b1), Wp1, r(bp1),
      r(ln_g2), r(ln_b2), Wp2, r(bp2), Wg1, r(bg1), wg2p, bg2p)
    return fused, nidx, w16, ent


def _call_classifier(fused, agg, Wu1, bu1, Wu2, bu2, Wc1, bc1,
                     bn_g, bn_b, bn_m, bn_v, Wc2, bc2):
    r = lambda a: a.reshape(1, -1)
    row_spec = lambda c: pl.BlockSpec((BLK, c), lambda i: (i, 0))
    full_spec = lambda rr, c: pl.BlockSpec((rr, c), lambda i: (0, 0))
    logits = pl.pallas_call(
        _classifier_body,
        grid=(NBLK,),
        in_specs=[
            row_spec(FUSED), row_spec(FUSED),
            full_spec(FUSED, FUSED), full_spec(1, FUSED),
            full_spec(FUSED, FUSED), full_spec(1, FUSED),
            full_spec(FUSED, H), full_spec(1, H),
            full_spec(1, H), full_spec(1, H), full_spec(1, H), full_spec(1, H),
            full_spec(H, 5), full_spec(1, 5),
        ],
        out_specs=[row_spec(5)],
        out_shape=[jax.ShapeDtypeStruct((B, 5), jnp.float32)],
    )(fused, agg, Wu1, r(bu1), Wu2, r(bu2), Wc1, r(bc1),
      r(bn_g), r(bn_b), r(bn_m), r(bn_v), Wc2, r(bc2))[0]
    return logits


def kernel(x, ln_g0, ln_b0, Wp0, bp0, ln_g1, ln_b1, Wp1, bp1, ln_g2, ln_b2,
           Wp2, bp2, Wg1, bg1, Wg2, bg2, Wu1, bu1, Wu2, bu2, Wc1, bc1,
           bn_g, bn_b, bn_m, bn_v, Wc2, bc2):
    fused, nidx, w16, ent = _call_fuse_knn(
        x, ln_g0, ln_b0, Wp0, bp0, ln_g1, ln_b1, Wp1, bp1,
        ln_g2, ln_b2, Wp2, bp2, Wg1, bg1, Wg2, bg2)
    agg = _make_sc_agg()(fused, nidx.reshape(-1), w16)
    logits = _call_classifier(fused, agg, Wu1, bu1, Wu2, bu2, Wc1, bc1,
                              bn_g, bn_b, bn_m, bn_v, Wc2, bc2)
    return (logits, ent[0, 0])
